# Initial kernel scaffold; baseline (speedup 1.0000x reference)
import jax
import jax.numpy as jnp
from jax.experimental import pallas as pl


def kernel(positional_indices, tables):
    raise NotImplementedError("write your pallas kernel here")


# speedup vs baseline: 2.4882x; 2.4882x over previous
import functools

import jax
import jax.numpy as jnp
from jax import lax
from jax.experimental import pallas as pl
from jax.experimental.pallas import tpu as pltpu, tpu_sc as plsc

D_MODEL = 1024
NUM_POSITIONS = 8
MAX_NODE = 15
BATCH = 16384
UNIT_D = D_MODEL // NUM_POSITIONS

B_FLAT = BATCH * NUM_POSITIONS
NW = 32
PER_W = B_FLAT // NW
CH = 128
NB = 4
GROUPS = PER_W // (NB * CH)


def _sc_gather(idx2, flat_table):
    mesh = plsc.VectorSubcoreMesh(core_axis_name="c", subcore_axis_name="s")

    @functools.partial(
        pl.kernel,
        out_type=jax.ShapeDtypeStruct((B_FLAT, UNIT_D), jnp.float32),
        mesh=mesh,
        scratch_types=[
            pltpu.VMEM((PER_W // CH, CH), jnp.int32),
            pltpu.VMEM((NB, CH, UNIT_D), jnp.float32),
            [pltpu.SemaphoreType.DMA] * NB,
        ],
    )
    def k(idx_hbm, tab_hbm, out_hbm, idx_v, rows_v, sems):
        wid = lax.axis_index("s") * 2 + lax.axis_index("c")
        row0 = wid * (PER_W // CH)
        base = wid * PER_W
        pltpu.sync_copy(idx_hbm.at[pl.ds(row0, PER_W // CH)], idx_v)

        def group(g):
            gbase = base + g * NB * CH
            gathers = []
            for b in range(NB):
                gathers.append(pltpu.async_copy(
                    tab_hbm.at[idx_v.at[g * NB + b]], rows_v.at[b], sems[b]))
            stores = []
            for b in range(NB):
                gathers[b].wait()
                stores.append(pltpu.async_copy(
                    rows_v.at[b],
                    out_hbm.at[pl.ds(pl.multiple_of(gbase + b * CH, CH), CH)],
                    sems[b]))
            for b in range(NB):
                stores[b].wait()

        pl.loop(0, GROUPS)(group)

    return k(idx2, flat_table)


def kernel(positional_indices, tables):
    idx = positional_indices.astype(jnp.int32)
    offs = (jnp.arange(NUM_POSITIONS, dtype=jnp.int32) * MAX_NODE)[:, None]
    fidx = (idx + offs).T.reshape(B_FLAT // CH, CH)
    flat_table = tables.reshape(NUM_POSITIONS * MAX_NODE, UNIT_D)
    out_flat = _sc_gather(fidx, flat_table)
    return out_flat.reshape(BATCH, D_MODEL)


# speedup vs baseline: 4.1197x; 1.6557x over previous
import functools

import jax
import jax.numpy as jnp
from jax import lax
from jax.experimental import pallas as pl
from jax.experimental.pallas import tpu as pltpu, tpu_sc as plsc

D_MODEL = 1024
NUM_POSITIONS = 8
MAX_NODE = 15
BATCH = 16384
UNIT_D = D_MODEL // NUM_POSITIONS

B_FLAT = BATCH * NUM_POSITIONS
NW = 32
PER_W = B_FLAT // NW
CH = 128
NB = 4
GROUPS = PER_W // (NB * CH)


def _sc_gather(idx2, flat_table):
    mesh = plsc.VectorSubcoreMesh(core_axis_name="c", subcore_axis_name="s")

    @functools.partial(
        pl.kernel,
        out_type=jax.ShapeDtypeStruct((B_FLAT, UNIT_D), jnp.float32),
        mesh=mesh,
        scratch_types=[
            pltpu.VMEM((PER_W // CH, CH), jnp.int32),
            pltpu.VMEM((NB, CH, UNIT_D), jnp.float32),
            pltpu.VMEM_SHARED((NUM_POSITIONS * MAX_NODE, UNIT_D), jnp.float32),
            [pltpu.SemaphoreType.DMA] * NB,
        ],
    )
    def k(idx_hbm, tab_hbm, out_hbm, idx_v, rows_v, tab_sp, sems):
        wid = lax.axis_index("s") * 2 + lax.axis_index("c")
        row0 = wid * (PER_W // CH)
        base = wid * PER_W

        @pl.when(lax.axis_index("s") == 0)
        def _copy_table():
            pltpu.sync_copy(tab_hbm, tab_sp)

        pltpu.sync_copy(idx_hbm.at[pl.ds(row0, PER_W // CH)], idx_v)
        plsc.subcore_barrier()

        def group(g):
            gbase = base + g * NB * CH
            gathers = []
            for b in range(NB):
                gathers.append(pltpu.async_copy(
                    tab_sp.at[idx_v.at[g * NB + b]], rows_v.at[b], sems[b]))
            stores = []
            for b in range(NB):
                gathers[b].wait()
                stores.append(pltpu.async_copy(
                    rows_v.at[b],
                    out_hbm.at[pl.ds(pl.multiple_of(gbase + b * CH, CH), CH)],
                    sems[b]))
            for b in range(NB):
                stores[b].wait()

        pl.loop(0, GROUPS)(group)

    return k(idx2, flat_table)


def kernel(positional_indices, tables):
    idx = positional_indices.astype(jnp.int32)
    offs = (jnp.arange(NUM_POSITIONS, dtype=jnp.int32) * MAX_NODE)[:, None]
    fidx = (idx + offs).T.reshape(B_FLAT // CH, CH)
    flat_table = tables.reshape(NUM_POSITIONS * MAX_NODE, UNIT_D)
    out_flat = _sc_gather(fidx, flat_table)
    return out_flat.reshape(BATCH, D_MODEL)


# speedup vs baseline: 10.0179x; 2.4317x over previous
import functools

import jax
import jax.numpy as jnp
from jax import lax
from jax.experimental import pallas as pl
from jax.experimental.pallas import tpu as pltpu, tpu_sc as plsc

D_MODEL = 1024
NUM_POSITIONS = 8
MAX_NODE = 15
BATCH = 16384
UNIT_D = D_MODEL // NUM_POSITIONS

NW = 32
CH = 128
BPW = BATCH // NW
CPW = BPW // CH
NB = 4
NCHUNK = CPW * NUM_POSITIONS
GROUPS = NCHUNK // NB


def _sc_gather(idx3, tables):
    mesh = plsc.VectorSubcoreMesh(core_axis_name="c", subcore_axis_name="s")

    @functools.partial(
        pl.kernel,
        out_type=jax.ShapeDtypeStruct((BATCH, D_MODEL), jnp.float32),
        mesh=mesh,
        scratch_types=[
            pltpu.VMEM((NUM_POSITIONS, CPW, CH), jnp.int32),
            pltpu.VMEM((NB, CH, UNIT_D), jnp.float32),
            pltpu.VMEM_SHARED((NUM_POSITIONS, 16, UNIT_D), jnp.float32),
            [pltpu.SemaphoreType.DMA] * NB,
        ],
    )
    def k(idx_hbm, tab_hbm, out_hbm, idx_v, rows_v, tab_sp, sems):
        wid = lax.axis_index("s") * 2 + lax.axis_index("c")
        b0 = wid * BPW

        @pl.when(lax.axis_index("s") == 0)
        def _copy_table():
            pltpu.sync_copy(tab_hbm, tab_sp)

        pltpu.sync_copy(idx_hbm.at[:, pl.ds(wid * CPW, CPW)], idx_v)
        plsc.subcore_barrier()

        def group(g):
            cb = g // 2
            i4 = (g % 2) * 4
            row = pl.multiple_of(b0 + cb * CH, CH)
            gathers = []
            for b in range(NB):
                i = i4 + b
                gathers.append(pltpu.async_copy(
                    tab_sp.at[i].at[idx_v.at[i, cb]], rows_v.at[b], sems[b]))
            stores = []
            for b in range(NB):
                i = i4 + b
                gathers[b].wait()
                stores.append(pltpu.async_copy(
                    rows_v.at[b],
                    out_hbm.at[pl.ds(row, CH),
                               pl.ds(pl.multiple_of(i * UNIT_D, UNIT_D), UNIT_D)],
                    sems[b]))
            for b in range(NB):
                stores[b].wait()

        pl.loop(0, GROUPS)(group)

    return k(idx3, tables)


def kernel(positional_indices, tables):
    idx3 = positional_indices.astype(jnp.int32).reshape(
        NUM_POSITIONS, BATCH // CH, CH)
    tab16 = jnp.pad(tables, ((0, 0), (0, 16 - MAX_NODE), (0, 0)))
    return _sc_gather(idx3, tab16)


# speedup vs baseline: 10.1811x; 1.0163x over previous
import functools

import jax
import jax.numpy as jnp
from jax import lax
from jax.experimental import pallas as pl
from jax.experimental.pallas import tpu as pltpu, tpu_sc as plsc

D_MODEL = 1024
NUM_POSITIONS = 8
MAX_NODE = 15
BATCH = 16384
UNIT_D = D_MODEL // NUM_POSITIONS

NW = 32
CH = 128
BPW = BATCH // NW
CPW = BPW // CH
NB = 4
NCHUNK = CPW * NUM_POSITIONS
GROUPS = NCHUNK // NB
PAD_NODE = 16


def _sc_gather(idx, tab16):
    mesh = plsc.VectorSubcoreMesh(core_axis_name="c", subcore_axis_name="s")

    @functools.partial(
        pl.kernel,
        out_type=jax.ShapeDtypeStruct((BATCH, D_MODEL), jnp.float32),
        mesh=mesh,
        scratch_types=[
            pltpu.VMEM((NUM_POSITIONS, BPW), jnp.int32),
            pltpu.VMEM((NB, CH, UNIT_D), jnp.float32),
            pltpu.VMEM_SHARED((NUM_POSITIONS, PAD_NODE, UNIT_D), jnp.float32),
            [pltpu.SemaphoreType.DMA] * NB,
        ],
    )
    def k(idx_hbm, tab_hbm, out_hbm, idx_v, rows_v, tab_sp, sems):
        wid = lax.axis_index("s") * 2 + lax.axis_index("c")
        b0 = wid * BPW

        @pl.when(lax.axis_index("s") == 0)
        def _copy_table():
            pltpu.sync_copy(tab_hbm, tab_sp)

        pltpu.sync_copy(idx_hbm.at[:, pl.ds(b0, BPW)], idx_v)
        plsc.subcore_barrier()

        def group(g):
            cb = g // 2
            i4 = (g % 2) * 4
            row = pl.multiple_of(b0 + cb * CH, CH)
            gathers = []
            for b in range(NB):
                i = i4 + b
                gathers.append(pltpu.async_copy(
                    tab_sp.at[i].at[idx_v.at[i, pl.ds(cb * CH, CH)]],
                    rows_v.at[b], sems[b]))
            stores = []
            for b in range(NB):
                i = i4 + b
                gathers[b].wait()
                stores.append(pltpu.async_copy(
                    rows_v.at[b],
                    out_hbm.at[pl.ds(row, CH),
                               pl.ds(pl.multiple_of(i * UNIT_D, UNIT_D), UNIT_D)],
                    sems[b]))
            for b in range(NB):
                stores[b].wait()

        pl.loop(0, GROUPS)(group)

    return k(idx, tab16)


def kernel(positional_indices, tables):
    idx = positional_indices.astype(jnp.int32)
    tab16 = jnp.pad(tables, ((0, 0), (0, PAD_NODE - MAX_NODE), (0, 0)))
    return _sc_gather(idx, tab16)
